# Initial kernel scaffold; baseline (speedup 1.0000x reference)
import jax
import jax.numpy as jnp
from jax.experimental import pallas as pl


def kernel(z, xyz, nbr_list, num_atoms, embed, W_in2f, b_in2f, W_f1, b_f1, W_f2, b_f2, W_o1, b_o1, W_o2, b_o2, W_r1, b_r1, W_r2, b_r2):
    raise NotImplementedError("write your pallas kernel here")


# speedup vs baseline: 1.6995x; 1.6995x over previous
import functools

import jax
import jax.numpy as jnp
from jax import lax
from jax.experimental import pallas as pl
from jax.experimental.pallas import tpu as pltpu
from jax.experimental.pallas import tpu_sc as plsc

NC = 2
NS = 16
NW = NC * NS
LANES = 16
CHUNK = 128


def _ssp(x):
    return jnp.maximum(x, 0.0) + jnp.log1p(jnp.exp(-jnp.abs(x))) - jnp.log(2.0)


def _sc_prep(z_pad, xs, ys, zs, src, dst, embed):
    n_pad = z_pad.shape[0]
    e_pad = src.shape[0]
    n_xyz = xs.shape[0]
    d_feat = embed.shape[1]
    rows_per_w = n_pad // NW
    g_chunk = 64
    e_per_w = e_pad // NW
    n_echunks = e_per_w // CHUNK

    mesh = plsc.VectorSubcoreMesh(core_axis_name="c", subcore_axis_name="s")

    @functools.partial(
        pl.kernel,
        out_type=(
            jax.ShapeDtypeStruct((n_pad, d_feat), jnp.float32),
            jax.ShapeDtypeStruct((e_pad,), jnp.float32),
        ),
        mesh=mesh,
        scratch_types=[
            pltpu.VMEM((g_chunk,), jnp.int32),
            pltpu.VMEM((g_chunk, d_feat), jnp.float32),
            pltpu.VMEM((n_xyz,), jnp.float32),
            pltpu.VMEM((n_xyz,), jnp.float32),
            pltpu.VMEM((n_xyz,), jnp.float32),
            pltpu.VMEM((CHUNK,), jnp.int32),
            pltpu.VMEM((CHUNK,), jnp.int32),
            pltpu.VMEM((CHUNK,), jnp.float32),
            pltpu.SemaphoreType.DMA,
        ],
        compiler_params=pltpu.CompilerParams(needs_layout_passes=False),
    )
    def prep(z_hbm, xs_hbm, ys_hbm, zs_hbm, src_hbm, dst_hbm, embed_hbm,
             r0_hbm, d2_hbm,
             zidx_v, rows_v, x_v, y_v, z_v, sidx_v, didx_v, d2_v, sem):
        wid = lax.axis_index("s") * NC + lax.axis_index("c")

        def embed_body(i, _):
            base = wid * rows_per_w + i * g_chunk
            pltpu.sync_copy(z_hbm.at[pl.ds(base, g_chunk)], zidx_v)
            pltpu.async_copy(embed_hbm.at[zidx_v], rows_v, sem).wait()
            pltpu.sync_copy(rows_v, r0_hbm.at[pl.ds(base, g_chunk)])
            return _

        lax.fori_loop(0, rows_per_w // g_chunk, embed_body, None)

        pltpu.sync_copy(xs_hbm, x_v)
        pltpu.sync_copy(ys_hbm, y_v)
        pltpu.sync_copy(zs_hbm, z_v)

        def edge_chunk(i, _):
            base = wid * e_per_w + i * CHUNK
            pltpu.sync_copy(src_hbm.at[pl.ds(base, CHUNK)], sidx_v)
            pltpu.sync_copy(dst_hbm.at[pl.ds(base, CHUNK)], didx_v)

            def vec_body(j, _):
                iv_s = sidx_v[pl.ds(j * LANES, LANES)]
                iv_d = didx_v[pl.ds(j * LANES, LANES)]
                dx = plsc.load_gather(x_v, [iv_s]) - plsc.load_gather(x_v, [iv_d])
                dy = plsc.load_gather(y_v, [iv_s]) - plsc.load_gather(y_v, [iv_d])
                dz = plsc.load_gather(z_v, [iv_s]) - plsc.load_gather(z_v, [iv_d])
                d2_v[pl.ds(j * LANES, LANES)] = dx * dx + dy * dy + dz * dz
                return _

            lax.fori_loop(0, CHUNK // LANES, vec_body, None, unroll=True)
            pltpu.sync_copy(d2_v, d2_hbm.at[pl.ds(base, CHUNK)])
            return _

        lax.fori_loop(0, n_echunks, edge_chunk, None)

    return prep(z_pad, xs, ys, zs, src, dst, embed)


def _tc_filter(d2, W_f1p, b_f1, W_f2, b_f2, e_real, cutoff, n_gauss):
    c_layers = W_f1p.shape[0]
    gp = W_f1p.shape[1]
    f_dim = W_f1p.shape[2]
    e_pad = d2.shape[0]
    BE = 512
    width = cutoff / (n_gauss - 1)

    def body(d2_ref, w1_ref, b1_ref, w2_ref, b2_ref, out_ref):
        eb = pl.program_id(1)
        d = jnp.sqrt(d2_ref[...] + 1e-12)
        offs = lax.broadcasted_iota(jnp.int32, (1, gp), 1).astype(jnp.float32) * width
        g = jnp.exp(-0.5 * jnp.square((d - offs) / width))
        u = _ssp(jnp.dot(g, w1_ref[0], preferred_element_type=jnp.float32)
                 + b1_ref[0])
        w = jnp.dot(u, w2_ref[0], preferred_element_type=jnp.float32) + b2_ref[0]
        rows = eb * BE + lax.broadcasted_iota(jnp.int32, (BE, 1), 0)
        out_ref[0] = jnp.where(rows < e_real, w, 0.0)

    return pl.pallas_call(
        body,
        grid=(c_layers, e_pad // BE),
        in_specs=[
            pl.BlockSpec((BE, 1), lambda c, e: (e, 0)),
            pl.BlockSpec((1, gp, f_dim), lambda c, e: (c, 0, 0)),
            pl.BlockSpec((1, 1, f_dim), lambda c, e: (c, 0, 0)),
            pl.BlockSpec((1, f_dim, f_dim), lambda c, e: (c, 0, 0)),
            pl.BlockSpec((1, 1, f_dim), lambda c, e: (c, 0, 0)),
        ],
        out_specs=pl.BlockSpec((1, BE, f_dim), lambda c, e: (c, e, 0)),
        out_shape=jax.ShapeDtypeStruct((c_layers, e_pad, f_dim), jnp.float32),
    )(d2, W_f1p, b_f1[:, None, :], W_f2, b_f2[:, None, :])


def _tc_dense(r, W, b, BN=400):
    n, d = r.shape
    f = W.shape[1]

    def body(r_ref, w_ref, b_ref, o_ref):
        o_ref[...] = (jnp.dot(r_ref[...], w_ref[...],
                              preferred_element_type=jnp.float32) + b_ref[...])

    return pl.pallas_call(
        body,
        grid=(n // BN,),
        in_specs=[
            pl.BlockSpec((BN, d), lambda i: (i, 0)),
            pl.BlockSpec((d, f), lambda i: (0, 0)),
            pl.BlockSpec((1, f), lambda i: (0, 0)),
        ],
        out_specs=pl.BlockSpec((BN, f), lambda i: (i, 0)),
        out_shape=jax.ShapeDtypeStruct((n, f), jnp.float32),
    )(r, W, b.reshape(1, f))


def _sc_message(h, w, src, dst, zeros_init):
    n, d_feat = h.shape
    e_pad = src.shape[0]
    e_per_w = e_pad // NW
    n_chunks = e_per_w // CHUNK
    n_acc = -(-n // (8 * NS)) * (8 * NS)
    rows_per_tile = n_acc // NS

    mesh = plsc.VectorSubcoreMesh(core_axis_name="c", subcore_axis_name="s")

    @functools.partial(
        pl.kernel,
        out_type=jax.ShapeDtypeStruct((NC, n_acc, d_feat), jnp.float32),
        mesh=mesh,
        scratch_types=[
            pltpu.VMEM_SHARED((n_acc, d_feat), jnp.float32),
            pltpu.VMEM((CHUNK,), jnp.int32),
            pltpu.VMEM((CHUNK,), jnp.int32),
            pltpu.VMEM((CHUNK, d_feat), jnp.float32),
            pltpu.VMEM((CHUNK, d_feat), jnp.float32),
            pltpu.SemaphoreType.DMA,
        ],
    )
    def msg(h_hbm, w_hbm, src_hbm, dst_hbm, zero_hbm, out_hbm,
            agg_sh, sidx_v, didx_v, h_v, w_v, sem):
        cid = lax.axis_index("c")
        sid = lax.axis_index("s")
        wid = sid * NC + cid

        pltpu.sync_copy(zero_hbm, agg_sh.at[pl.ds(sid * rows_per_tile,
                                                  rows_per_tile)])
        plsc.subcore_barrier()

        def chunk_body(i, _):
            base = wid * e_per_w + i * CHUNK
            pltpu.sync_copy(src_hbm.at[pl.ds(base, CHUNK)], sidx_v)
            pltpu.sync_copy(dst_hbm.at[pl.ds(base, CHUNK)], didx_v)
            gat = pltpu.async_copy(h_hbm.at[didx_v], h_v, sem)
            pltpu.sync_copy(w_hbm.at[pl.ds(base, CHUNK)], w_v)
            gat.wait()

            def mul_body(e, _):
                for k in range(d_feat // LANES):
                    sl = pl.ds(k * LANES, LANES)
                    h_v[e, sl] = h_v[e, sl] * w_v[e, sl]
                return _

            lax.fori_loop(0, CHUNK, mul_body, None)
            pltpu.sync_copy(h_v, agg_sh.at[sidx_v], add=True)
            return _

        lax.fori_loop(0, n_chunks, chunk_body, None)
        plsc.subcore_barrier()
        row0 = sid * rows_per_tile
        pltpu.sync_copy(agg_sh.at[pl.ds(row0, rows_per_tile)],
                        out_hbm.at[cid, pl.ds(row0, rows_per_tile)])

    return msg(h, w, src, dst, zeros_init)


def _tc_update(partials, r, W_o1, b_o1, W_o2, b_o2, BN=400):
    n, d = r.shape

    def body(p_ref, r_ref, w1_ref, b1_ref, w2_ref, b2_ref, o_ref):
        agg = p_ref[0] + p_ref[1]
        t = _ssp(jnp.dot(agg, w1_ref[...], preferred_element_type=jnp.float32)
                 + b1_ref[...])
        dr = jnp.dot(t, w2_ref[...], preferred_element_type=jnp.float32) + b2_ref[...]
        o_ref[...] = r_ref[...] + dr

    return pl.pallas_call(
        body,
        grid=(n // BN,),
        in_specs=[
            pl.BlockSpec((2, BN, d), lambda i: (0, i, 0)),
            pl.BlockSpec((BN, d), lambda i: (i, 0)),
            pl.BlockSpec((d, d), lambda i: (0, 0)),
            pl.BlockSpec((1, d), lambda i: (0, 0)),
            pl.BlockSpec((d, d), lambda i: (0, 0)),
            pl.BlockSpec((1, d), lambda i: (0, 0)),
        ],
        out_specs=pl.BlockSpec((BN, d), lambda i: (i, 0)),
        out_shape=jax.ShapeDtypeStruct((n, d), jnp.float32),
    )(partials, r, W_o1, b_o1.reshape(1, d), W_o2, b_o2.reshape(1, d))


def _tc_readout(r, W_r1, b_r1, w_r2_row, b_r2, BN=400):
    n, d = r.shape
    dh = W_r1.shape[1]

    def body(r_ref, w1_ref, b1_ref, w2_ref, b2_ref, o_ref):
        i = pl.program_id(0)
        t = _ssp(jnp.dot(r_ref[...], w1_ref[...],
                         preferred_element_type=jnp.float32) + b1_ref[...])
        s = jnp.sum(t * w2_ref[...]) + BN * b2_ref[0, 0]

        @pl.when(i == 0)
        def _():
            o_ref[...] = jnp.zeros((1, 1), jnp.float32)

        o_ref[...] = o_ref[...] + s

    return pl.pallas_call(
        body,
        grid=(n // BN,),
        in_specs=[
            pl.BlockSpec((BN, d), lambda i: (i, 0)),
            pl.BlockSpec((d, dh), lambda i: (0, 0)),
            pl.BlockSpec((1, dh), lambda i: (0, 0)),
            pl.BlockSpec((1, dh), lambda i: (0, 0)),
            pl.BlockSpec((1, 1), lambda i: (0, 0)),
        ],
        out_specs=pl.BlockSpec((1, 1), lambda i: (0, 0)),
        out_shape=jax.ShapeDtypeStruct((1, 1), jnp.float32),
    )(r, W_r1, b_r1.reshape(1, dh), w_r2_row, b_r2.reshape(1, 1))


def kernel(z, xyz, nbr_list, num_atoms, embed,
           W_in2f, b_in2f, W_f1, b_f1, W_f2, b_f2,
           W_o1, b_o1, W_o2, b_o2, W_r1, b_r1, W_r2, b_r2):
    cutoff = 5.0
    n = z.shape[0]
    e = nbr_list.shape[0]
    c_layers = W_in2f.shape[0]
    n_gauss = W_f1.shape[1]
    d_feat = embed.shape[1]

    e_pad = -(-e // (NW * CHUNK)) * (NW * CHUNK)
    n_pad = -(-n // (NW * 64)) * (NW * 64)

    nbr_pad = jnp.pad(nbr_list.astype(jnp.int32), ((0, e_pad - e), (0, 0)))
    src = nbr_pad[:, 0]
    dst = nbr_pad[:, 1]
    z_pad = jnp.pad(z.astype(jnp.int32), (0, n_pad - n))

    r0_pad, d2 = _sc_prep(z_pad, xyz[:, 0], xyz[:, 1], xyz[:, 2],
                          src, dst, embed)
    r = r0_pad[:n]
    d2 = d2.reshape(e_pad, 1)

    gp = -(-n_gauss // 8) * 8
    W_f1p = jnp.pad(W_f1, ((0, 0), (0, gp - n_gauss), (0, 0)))
    w_all = _tc_filter(d2, W_f1p, b_f1, W_f2, b_f2, e, cutoff, n_gauss)

    n_acc = -(-n // (8 * NS)) * (8 * NS)
    zeros_init = jnp.zeros((n_acc // NS, d_feat), jnp.float32)
    for i in range(c_layers):
        h = _tc_dense(r, W_in2f[i], b_in2f[i])
        partials = _sc_message(h, w_all[i], src, dst, zeros_init)[:, :n]
        r = _tc_update(partials, r, W_o1[i], b_o1[i], W_o2[i], b_o2[i])

    energy = _tc_readout(r, W_r1, b_r1, W_r2.T, b_r2)
    return energy.reshape(1)


# speedup vs baseline: 2.1222x; 1.2487x over previous
import functools

import jax
import jax.numpy as jnp
from jax import lax
from jax.experimental import pallas as pl
from jax.experimental.pallas import tpu as pltpu
from jax.experimental.pallas import tpu_sc as plsc

NC = 2
NS = 16
NW = NC * NS
LANES = 16
CHUNK = 128


def _ssp(x):
    return jnp.maximum(x, 0.0) + jnp.log1p(jnp.exp(-jnp.abs(x))) - jnp.log(2.0)


def _sc_prep(z_pad, xs, ys, zs, src, dst, embed):
    n_pad = z_pad.shape[0]
    e_pad = src.shape[0]
    n_xyz = xs.shape[0]
    d_feat = embed.shape[1]
    rows_per_w = n_pad // NW
    g_chunk = 64
    e_per_w = e_pad // NW
    n_echunks = e_per_w // CHUNK

    mesh = plsc.VectorSubcoreMesh(core_axis_name="c", subcore_axis_name="s")

    @functools.partial(
        pl.kernel,
        out_type=(
            jax.ShapeDtypeStruct((n_pad, d_feat), jnp.float32),
            jax.ShapeDtypeStruct((e_pad,), jnp.float32),
        ),
        mesh=mesh,
        scratch_types=[
            pltpu.VMEM((g_chunk,), jnp.int32),
            pltpu.VMEM((g_chunk, d_feat), jnp.float32),
            pltpu.VMEM((n_xyz,), jnp.float32),
            pltpu.VMEM((n_xyz,), jnp.float32),
            pltpu.VMEM((n_xyz,), jnp.float32),
            pltpu.VMEM((CHUNK,), jnp.int32),
            pltpu.VMEM((CHUNK,), jnp.int32),
            pltpu.VMEM((CHUNK,), jnp.float32),
            pltpu.SemaphoreType.DMA,
        ],
        compiler_params=pltpu.CompilerParams(needs_layout_passes=False),
    )
    def prep(z_hbm, xs_hbm, ys_hbm, zs_hbm, src_hbm, dst_hbm, embed_hbm,
             r0_hbm, d2_hbm,
             zidx_v, rows_v, x_v, y_v, z_v, sidx_v, didx_v, d2_v, sem):
        wid = lax.axis_index("s") * NC + lax.axis_index("c")

        def embed_body(i, _):
            base = wid * rows_per_w + i * g_chunk
            pltpu.sync_copy(z_hbm.at[pl.ds(base, g_chunk)], zidx_v)
            pltpu.async_copy(embed_hbm.at[zidx_v], rows_v, sem).wait()
            pltpu.sync_copy(rows_v, r0_hbm.at[pl.ds(base, g_chunk)])
            return _

        lax.fori_loop(0, rows_per_w // g_chunk, embed_body, None)

        pltpu.sync_copy(xs_hbm, x_v)
        pltpu.sync_copy(ys_hbm, y_v)
        pltpu.sync_copy(zs_hbm, z_v)
        nmax = n_xyz - 1

        def edge_chunk(i, _):
            base = wid * e_per_w + i * CHUNK
            pltpu.sync_copy(src_hbm.at[pl.ds(base, CHUNK)], sidx_v)
            pltpu.sync_copy(dst_hbm.at[pl.ds(base, CHUNK)], didx_v)

            def vec_body(j, _):
                iv_s = jnp.minimum(sidx_v[pl.ds(j * LANES, LANES)], nmax)
                iv_d = jnp.minimum(didx_v[pl.ds(j * LANES, LANES)], nmax)
                dx = plsc.load_gather(x_v, [iv_s]) - plsc.load_gather(x_v, [iv_d])
                dy = plsc.load_gather(y_v, [iv_s]) - plsc.load_gather(y_v, [iv_d])
                dz = plsc.load_gather(z_v, [iv_s]) - plsc.load_gather(z_v, [iv_d])
                d2_v[pl.ds(j * LANES, LANES)] = dx * dx + dy * dy + dz * dz
                return _

            lax.fori_loop(0, CHUNK // LANES, vec_body, None, unroll=True)
            pltpu.sync_copy(d2_v, d2_hbm.at[pl.ds(base, CHUNK)])
            return _

        lax.fori_loop(0, n_echunks, edge_chunk, None)

    return prep(z_pad, xs, ys, zs, src, dst, embed)


def _tc_filter(d2, W_f1p, b_f1, W_f2, b_f2, cutoff, n_gauss):
    c_layers = W_f1p.shape[0]
    gp = W_f1p.shape[1]
    f_dim = W_f1p.shape[2]
    dh = f_dim // NC
    e_pad = d2.shape[0]
    BE = 512
    width = cutoff / (n_gauss - 1)

    def body(d2_ref, w1_ref, b1_ref, w2_ref, b2_ref, out_ref):
        d = jnp.sqrt(d2_ref[...] + 1e-12)
        offs = lax.broadcasted_iota(jnp.int32, (1, gp), 1).astype(jnp.float32) * width
        g = jnp.exp(-0.5 * jnp.square((d - offs) / width))
        for c in range(c_layers):
            u = _ssp(jnp.dot(g, w1_ref[c], preferred_element_type=jnp.float32)
                     + b1_ref[c])
            w = (jnp.dot(u, w2_ref[c], preferred_element_type=jnp.float32)
                 + b2_ref[c])
            for h in range(NC):
                out_ref[c, h] = w[:, h * dh:(h + 1) * dh]

    return pl.pallas_call(
        body,
        grid=(e_pad // BE,),
        in_specs=[
            pl.BlockSpec((BE, 1), lambda e: (e, 0)),
            pl.BlockSpec((c_layers, gp, f_dim), lambda e: (0, 0, 0)),
            pl.BlockSpec((c_layers, 1, f_dim), lambda e: (0, 0, 0)),
            pl.BlockSpec((c_layers, f_dim, f_dim), lambda e: (0, 0, 0)),
            pl.BlockSpec((c_layers, 1, f_dim), lambda e: (0, 0, 0)),
        ],
        out_specs=pl.BlockSpec((c_layers, NC, BE, dh), lambda e: (0, 0, e, 0)),
        out_shape=jax.ShapeDtypeStruct((c_layers, NC, e_pad, dh), jnp.float32),
    )(d2, W_f1p, b_f1[:, None, :], W_f2, b_f2[:, None, :])


def _tc_dense(r, W, b, BN=400):
    n, d = r.shape
    f = W.shape[1]
    dh = f // NC

    def body(r_ref, w_ref, b_ref, o_ref):
        h = (jnp.dot(r_ref[...], w_ref[...],
                     preferred_element_type=jnp.float32) + b_ref[...])
        for c in range(NC):
            o_ref[c] = h[:, c * dh:(c + 1) * dh]

    return pl.pallas_call(
        body,
        grid=(n // BN,),
        in_specs=[
            pl.BlockSpec((BN, d), lambda i: (i, 0)),
            pl.BlockSpec((d, f), lambda i: (0, 0)),
            pl.BlockSpec((1, f), lambda i: (0, 0)),
        ],
        out_specs=pl.BlockSpec((NC, BN, dh), lambda i: (0, i, 0)),
        out_shape=jax.ShapeDtypeStruct((NC, n, dh), jnp.float32),
    )(r, W, b.reshape(1, f))


def _sc_message(h2, w5, src3, dst3, zeros_init, n_acc):
    _, n, dh = h2.shape
    n_chunks = src3.shape[1]
    rows_per_tile = n_acc // NS

    mesh = plsc.VectorSubcoreMesh(core_axis_name="c", subcore_axis_name="s")

    @functools.partial(
        pl.kernel,
        out_type=jax.ShapeDtypeStruct((NC, n_acc, dh), jnp.float32),
        mesh=mesh,
        scratch_types=[
            pltpu.VMEM_SHARED((n_acc, dh), jnp.float32),
            pltpu.VMEM((n_chunks, CHUNK), jnp.int32),
            pltpu.VMEM((n_chunks, CHUNK), jnp.int32),
            pltpu.VMEM((CHUNK, dh), jnp.float32),
            pltpu.VMEM((CHUNK, dh), jnp.float32),
            pltpu.VMEM((CHUNK, dh), jnp.float32),
            pltpu.VMEM((CHUNK, dh), jnp.float32),
            pltpu.SemaphoreType.DMA,
            pltpu.SemaphoreType.DMA,
        ],
        compiler_params=pltpu.CompilerParams(use_tc_tiling_on_sc=False),
    )
    def msg(h_hbm, w_hbm, src_hbm, dst_hbm, zero_hbm, out_hbm,
            agg_sh, sidx_all, didx_all, h0_v, w0_v, h1_v, w1_v,
            b0_sem, b1_sem):
        cid = lax.axis_index("c")
        sid = lax.axis_index("s")

        pltpu.sync_copy(zero_hbm, agg_sh.at[pl.ds(sid * rows_per_tile,
                                                  rows_per_tile)])
        pltpu.sync_copy(src_hbm.at[sid], sidx_all)
        pltpu.sync_copy(dst_hbm.at[sid], didx_all)
        plsc.subcore_barrier()

        h_view = h_hbm.at[cid]
        bufs = ((h0_v, w0_v, b0_sem), (h1_v, w1_v, b1_sem))

        def start(i, h_v, w_v, b_sem):
            pltpu.async_copy(h_view.at[didx_all.at[i]], h_v, b_sem)
            pltpu.async_copy(w_hbm.at[cid, sid, i], w_v, b_sem)

        def process(i, h_v, w_v, b_sem):
            pltpu.make_async_copy(h_view.at[didx_all.at[i]], h_v, b_sem).wait()
            pltpu.make_async_copy(w_hbm.at[cid, sid, i], w_v, b_sem).wait()

            def mul_body(e, _):
                for k in range(dh // LANES):
                    sl = pl.ds(k * LANES, LANES)
                    h_v[e, sl] = h_v[e, sl] * w_v[e, sl]
                return _

            lax.fori_loop(0, CHUNK, mul_body, None)
            pltpu.sync_copy(h_v, agg_sh.at[sidx_all.at[i]], add=True)

        start(0, *bufs[0])
        start(1, *bufs[1])

        def pair_body(t, _):
            for b in range(2):
                i = 2 * t + b
                process(i, *bufs[b])

                @pl.when(i + 2 < n_chunks)
                def _():
                    start(i + 2, *bufs[b])
            return _

        lax.fori_loop(0, n_chunks // 2, pair_body, None)
        plsc.subcore_barrier()
        row0 = sid * rows_per_tile
        pltpu.sync_copy(agg_sh.at[pl.ds(row0, rows_per_tile)],
                        out_hbm.at[cid, pl.ds(row0, rows_per_tile)])

    return msg(h2, w5, src3, dst3, zeros_init)


def _tc_update(partials, r, W_o1, b_o1, W_o2, b_o2, BN=400):
    n, d = r.shape
    dh = d // NC

    def body(p_ref, r_ref, w1_ref, b1_ref, w2_ref, b2_ref, o_ref):
        agg = jnp.concatenate([p_ref[0], p_ref[1]], axis=1)
        t = _ssp(jnp.dot(agg, w1_ref[...], preferred_element_type=jnp.float32)
                 + b1_ref[...])
        dr = jnp.dot(t, w2_ref[...], preferred_element_type=jnp.float32) + b2_ref[...]
        o_ref[...] = r_ref[...] + dr

    return pl.pallas_call(
        body,
        grid=(n // BN,),
        in_specs=[
            pl.BlockSpec((NC, BN, dh), lambda i: (0, i, 0)),
            pl.BlockSpec((BN, d), lambda i: (i, 0)),
            pl.BlockSpec((d, d), lambda i: (0, 0)),
            pl.BlockSpec((1, d), lambda i: (0, 0)),
            pl.BlockSpec((d, d), lambda i: (0, 0)),
            pl.BlockSpec((1, d), lambda i: (0, 0)),
        ],
        out_specs=pl.BlockSpec((BN, d), lambda i: (i, 0)),
        out_shape=jax.ShapeDtypeStruct((n, d), jnp.float32),
    )(partials, r, W_o1, b_o1.reshape(1, d), W_o2, b_o2.reshape(1, d))


def _tc_readout(r, W_r1, b_r1, w_r2_row, b_r2, BN=400):
    n, d = r.shape
    dh = W_r1.shape[1]

    def body(r_ref, w1_ref, b1_ref, w2_ref, b2_ref, o_ref):
        i = pl.program_id(0)
        t = _ssp(jnp.dot(r_ref[...], w1_ref[...],
                         preferred_element_type=jnp.float32) + b1_ref[...])
        s = jnp.sum(t * w2_ref[...]) + BN * b2_ref[0, 0]

        @pl.when(i == 0)
        def _():
            o_ref[...] = jnp.zeros((1, 1), jnp.float32)

        o_ref[...] = o_ref[...] + s

    return pl.pallas_call(
        body,
        grid=(n // BN,),
        in_specs=[
            pl.BlockSpec((BN, d), lambda i: (i, 0)),
            pl.BlockSpec((d, dh), lambda i: (0, 0)),
            pl.BlockSpec((1, dh), lambda i: (0, 0)),
            pl.BlockSpec((1, dh), lambda i: (0, 0)),
            pl.BlockSpec((1, 1), lambda i: (0, 0)),
        ],
        out_specs=pl.BlockSpec((1, 1), lambda i: (0, 0)),
        out_shape=jax.ShapeDtypeStruct((1, 1), jnp.float32),
    )(r, W_r1, b_r1.reshape(1, dh), w_r2_row, b_r2.reshape(1, 1))


def kernel(z, xyz, nbr_list, num_atoms, embed,
           W_in2f, b_in2f, W_f1, b_f1, W_f2, b_f2,
           W_o1, b_o1, W_o2, b_o2, W_r1, b_r1, W_r2, b_r2):
    cutoff = 5.0
    n = z.shape[0]
    e = nbr_list.shape[0]
    c_layers = W_in2f.shape[0]
    n_gauss = W_f1.shape[1]
    d_feat = embed.shape[1]
    dh = d_feat // NC

    e_pad = -(-e // (NS * CHUNK * 2)) * (NS * CHUNK * 2)
    n_pad = -(-n // (NW * 64)) * (NW * 64)
    n_acc = -(-(n + 1) // (8 * NS)) * (8 * NS)

    nbr = nbr_list.astype(jnp.int32)
    src = jnp.concatenate([nbr[:, 0], jnp.full((e_pad - e,), n, jnp.int32)])
    dst = jnp.concatenate([nbr[:, 1], jnp.zeros((e_pad - e,), jnp.int32)])
    z_pad = jnp.pad(z.astype(jnp.int32), (0, n_pad - n))

    r0_pad, d2 = _sc_prep(z_pad, xyz[:, 0], xyz[:, 1], xyz[:, 2],
                          src, dst, embed)
    r = r0_pad[:n]
    d2 = d2.reshape(e_pad, 1)

    gp = -(-n_gauss // 8) * 8
    W_f1p = jnp.pad(W_f1, ((0, 0), (0, gp - n_gauss), (0, 0)))
    w_all = _tc_filter(d2, W_f1p, b_f1, W_f2, b_f2, cutoff, n_gauss)

    n_chunks = e_pad // (NS * CHUNK)
    src3 = src.reshape(NS, n_chunks, CHUNK)
    dst3 = dst.reshape(NS, n_chunks, CHUNK)
    zeros_init = jnp.zeros((n_acc // NS, dh), jnp.float32)
    for i in range(c_layers):
        h2 = _tc_dense(r, W_in2f[i], b_in2f[i])
        w5 = w_all[i].reshape(NC, NS, n_chunks, CHUNK, dh)
        partials = _sc_message(h2, w5, src3, dst3, zeros_init, n_acc)[:, :n]
        r = _tc_update(partials, r, W_o1[i], b_o1[i], W_o2[i], b_o2[i])

    energy = _tc_readout(r, W_r1, b_r1, W_r2.T, b_r2)
    return energy.reshape(1)


# speedup vs baseline: 2.5810x; 1.2162x over previous
import functools

import jax
import jax.numpy as jnp
from jax import lax
from jax.experimental import pallas as pl
from jax.experimental.pallas import tpu as pltpu
from jax.experimental.pallas import tpu_sc as plsc

NC = 2
NS = 16
NW = NC * NS
LANES = 16
CHUNK = 128


def _dot(a, b):
    return jnp.dot(a.astype(jnp.bfloat16), b.astype(jnp.bfloat16),
                   preferred_element_type=jnp.float32)


def _ssp(x):
    return jnp.maximum(x, 0.0) + jnp.log1p(jnp.exp(-jnp.abs(x))) - jnp.log(2.0)


def _tc_embed_table(embed, W, b):
    v, d = embed.shape
    f = W.shape[1]
    dh = f // NC

    def body(e_ref, w_ref, b_ref, o_ref):
        h = _dot(e_ref[...], w_ref[...]) + b_ref[...]
        for c in range(NC):
            o_ref[c] = h[:, c * dh:(c + 1) * dh]

    return pl.pallas_call(
        body,
        out_shape=jax.ShapeDtypeStruct((NC, v, dh), jnp.float32),
    )(embed, W, b.reshape(1, f))


def _sc_prep(z_pad, xs, ys, zs, src, dst, embed, hWs):
    n_pad = z_pad.shape[0]
    e_pad = src.shape[0]
    n_xyz = xs.shape[0]
    d_feat = embed.shape[1]
    dh = d_feat // NC
    rows_per_w = n_pad // NW
    g_chunk = 64
    e_per_w = e_pad // NW
    n_echunks = e_per_w // CHUNK

    mesh = plsc.VectorSubcoreMesh(core_axis_name="c", subcore_axis_name="s")

    @functools.partial(
        pl.kernel,
        out_type=(
            jax.ShapeDtypeStruct((n_pad, d_feat), jnp.float32),
            jax.ShapeDtypeStruct((NC, n_pad, dh), jnp.float32),
            jax.ShapeDtypeStruct((e_pad,), jnp.float32),
        ),
        mesh=mesh,
        scratch_types=[
            pltpu.VMEM((g_chunk,), jnp.int32),
            pltpu.VMEM((g_chunk, d_feat), jnp.float32),
            pltpu.VMEM((g_chunk, dh), jnp.float32),
            pltpu.VMEM((n_xyz,), jnp.float32),
            pltpu.VMEM((n_xyz,), jnp.float32),
            pltpu.VMEM((n_xyz,), jnp.float32),
            pltpu.VMEM((CHUNK,), jnp.int32),
            pltpu.VMEM((CHUNK,), jnp.int32),
            pltpu.VMEM((CHUNK,), jnp.float32),
            pltpu.SemaphoreType.DMA,
        ],
        compiler_params=pltpu.CompilerParams(needs_layout_passes=False,
                                             use_tc_tiling_on_sc=False),
    )
    def prep(z_hbm, xs_hbm, ys_hbm, zs_hbm, src_hbm, dst_hbm, embed_hbm,
             hw_hbm, r0_hbm, h0_hbm, d2_hbm,
             zidx_v, rows_v, rowh_v, x_v, y_v, z_v, sidx_v, didx_v, d2_v, sem):
        wid = lax.axis_index("s") * NC + lax.axis_index("c")

        def embed_body(i, _):
            base = wid * rows_per_w + i * g_chunk
            pltpu.sync_copy(z_hbm.at[pl.ds(base, g_chunk)], zidx_v)
            pltpu.async_copy(embed_hbm.at[zidx_v], rows_v, sem).wait()
            pltpu.sync_copy(rows_v, r0_hbm.at[pl.ds(base, g_chunk)])
            for c in range(NC):
                pltpu.async_copy(hw_hbm.at[c].at[zidx_v], rowh_v, sem).wait()
                pltpu.sync_copy(rowh_v, h0_hbm.at[c, pl.ds(base, g_chunk)])
            return _

        lax.fori_loop(0, rows_per_w // g_chunk, embed_body, None)

        pltpu.sync_copy(xs_hbm, x_v)
        pltpu.sync_copy(ys_hbm, y_v)
        pltpu.sync_copy(zs_hbm, z_v)
        nmax = n_xyz - 1

        def edge_chunk(i, _):
            base = wid * e_per_w + i * CHUNK
            pltpu.sync_copy(src_hbm.at[pl.ds(base, CHUNK)], sidx_v)
            pltpu.sync_copy(dst_hbm.at[pl.ds(base, CHUNK)], didx_v)

            def vec_body(j, _):
                iv_s = jnp.minimum(sidx_v[pl.ds(j * LANES, LANES)], nmax)
                iv_d = jnp.minimum(didx_v[pl.ds(j * LANES, LANES)], nmax)
                dx = plsc.load_gather(x_v, [iv_s]) - plsc.load_gather(x_v, [iv_d])
                dy = plsc.load_gather(y_v, [iv_s]) - plsc.load_gather(y_v, [iv_d])
                dz = plsc.load_gather(z_v, [iv_s]) - plsc.load_gather(z_v, [iv_d])
                d2_v[pl.ds(j * LANES, LANES)] = dx * dx + dy * dy + dz * dz
                return _

            lax.fori_loop(0, CHUNK // LANES, vec_body, None, unroll=True)
            pltpu.sync_copy(d2_v, d2_hbm.at[pl.ds(base, CHUNK)])
            return _

        lax.fori_loop(0, n_echunks, edge_chunk, None)

    return prep(z_pad, xs, ys, zs, src, dst, embed, hWs)


def _tc_filter(d2, W_f1p, b_f1, W_f2, b_f2, cutoff, n_gauss):
    c_layers = W_f1p.shape[0]
    gp = W_f1p.shape[1]
    f_dim = W_f1p.shape[2]
    dh = f_dim // NC
    e_pad = d2.shape[0]
    BE = 1024
    width = cutoff / (n_gauss - 1)

    def body(d2_ref, w1_ref, b1_ref, w2_ref, b2_ref, out_ref):
        d = jnp.sqrt(d2_ref[...] + 1e-12)
        offs = lax.broadcasted_iota(jnp.int32, (1, gp), 1).astype(jnp.float32) * width
        g = jnp.exp(-0.5 * jnp.square((d - offs) / width))
        for c in range(c_layers):
            u = _ssp(_dot(g, w1_ref[c]) + b1_ref[c])
            w = _dot(u, w2_ref[c]) + b2_ref[c]
            for h in range(NC):
                out_ref[c, h] = w[:, h * dh:(h + 1) * dh]

    return pl.pallas_call(
        body,
        grid=(e_pad // BE,),
        in_specs=[
            pl.BlockSpec((BE, 1), lambda e: (e, 0)),
            pl.BlockSpec((c_layers, gp, f_dim), lambda e: (0, 0, 0)),
            pl.BlockSpec((c_layers, 1, f_dim), lambda e: (0, 0, 0)),
            pl.BlockSpec((c_layers, f_dim, f_dim), lambda e: (0, 0, 0)),
            pl.BlockSpec((c_layers, 1, f_dim), lambda e: (0, 0, 0)),
        ],
        out_specs=pl.BlockSpec((c_layers, NC, BE, dh), lambda e: (0, 0, e, 0)),
        out_shape=jax.ShapeDtypeStruct((c_layers, NC, e_pad, dh), jnp.float32),
    )(d2, W_f1p, b_f1[:, None, :], W_f2, b_f2[:, None, :])


def _sc_message(h2, w_all6, layer, src3, dst3, zeros_init, n_acc):
    _, n, dh = h2.shape
    n_chunks = src3.shape[1]
    rows_per_tile = n_acc // NS

    mesh = plsc.VectorSubcoreMesh(core_axis_name="c", subcore_axis_name="s")

    @functools.partial(
        pl.kernel,
        out_type=jax.ShapeDtypeStruct((NC, n_acc, dh), jnp.float32),
        mesh=mesh,
        scratch_types=[
            pltpu.VMEM_SHARED((n_acc, dh), jnp.float32),
            pltpu.VMEM((n_chunks, CHUNK), jnp.int32),
            pltpu.VMEM((n_chunks, CHUNK), jnp.int32),
            pltpu.VMEM((CHUNK, dh), jnp.float32),
            pltpu.VMEM((CHUNK, dh), jnp.float32),
            pltpu.VMEM((CHUNK, dh), jnp.float32),
            pltpu.VMEM((CHUNK, dh), jnp.float32),
            pltpu.SemaphoreType.DMA,
            pltpu.SemaphoreType.DMA,
        ],
        compiler_params=pltpu.CompilerParams(use_tc_tiling_on_sc=False),
    )
    def msg(h_hbm, w_hbm, src_hbm, dst_hbm, zero_hbm, out_hbm,
            agg_sh, sidx_all, didx_all, h0_v, w0_v, h1_v, w1_v,
            b0_sem, b1_sem):
        cid = lax.axis_index("c")
        sid = lax.axis_index("s")

        pltpu.sync_copy(zero_hbm, agg_sh.at[pl.ds(sid * rows_per_tile,
                                                  rows_per_tile)])
        pltpu.sync_copy(src_hbm.at[sid], sidx_all)
        pltpu.sync_copy(dst_hbm.at[sid], didx_all)
        plsc.subcore_barrier()

        h_view = h_hbm.at[cid]
        w_view = w_hbm.at[layer]
        bufs = ((h0_v, w0_v, b0_sem), (h1_v, w1_v, b1_sem))

        def start(i, h_v, w_v, b_sem):
            pltpu.async_copy(h_view.at[didx_all.at[i]], h_v, b_sem)
            pltpu.async_copy(w_view.at[cid, sid, i], w_v, b_sem)

        def process(i, h_v, w_v, b_sem):
            pltpu.make_async_copy(h_view.at[didx_all.at[i]], h_v, b_sem).wait()
            pltpu.make_async_copy(w_view.at[cid, sid, i], w_v, b_sem).wait()

            def mul_body(e, _):
                for k in range(dh // LANES):
                    sl = pl.ds(k * LANES, LANES)
                    h_v[e, sl] = h_v[e, sl] * w_v[e, sl]
                return _

            lax.fori_loop(0, CHUNK, mul_body, None)
            pltpu.sync_copy(h_v, agg_sh.at[sidx_all.at[i]], add=True)

        start(0, *bufs[0])
        start(1, *bufs[1])

        def pair_body(t, _):
            for b in range(2):
                i = 2 * t + b
                process(i, *bufs[b])

                @pl.when(i + 2 < n_chunks)
                def _():
                    start(i + 2, *bufs[b])
            return _

        lax.fori_loop(0, n_chunks // 2, pair_body, None)
        plsc.subcore_barrier()
        row0 = sid * rows_per_tile
        pltpu.sync_copy(agg_sh.at[pl.ds(row0, rows_per_tile)],
                        out_hbm.at[cid, pl.ds(row0, rows_per_tile)])

    return msg(h2, w_all6, src3, dst3, zeros_init)


def _tc_update_dense(partials, r, W_o1, b_o1, W_o2, b_o2, W_in, b_in,
                     n, BN=400):
    d = r.shape[1]
    dh = d // NC
    n_acc = partials.shape[1]

    def body(p_ref, r_ref, w1_ref, b1_ref, w2_ref, b2_ref, wi_ref, bi_ref,
             ro_ref, ho_ref):
        agg = jnp.concatenate([p_ref[0], p_ref[1]], axis=1)
        t = _ssp(_dot(agg, w1_ref[...]) + b1_ref[...])
        dr = _dot(t, w2_ref[...]) + b2_ref[...]
        rn = r_ref[...] + dr
        ro_ref[...] = rn
        h = _dot(rn, wi_ref[...]) + bi_ref[...]
        for c in range(NC):
            ho_ref[c] = h[:, c * dh:(c + 1) * dh]

    return pl.pallas_call(
        body,
        grid=(n // BN,),
        in_specs=[
            pl.BlockSpec((NC, BN, dh), lambda i: (0, i, 0)),
            pl.BlockSpec((BN, d), lambda i: (i, 0)),
            pl.BlockSpec((d, d), lambda i: (0, 0)),
            pl.BlockSpec((1, d), lambda i: (0, 0)),
            pl.BlockSpec((d, d), lambda i: (0, 0)),
            pl.BlockSpec((1, d), lambda i: (0, 0)),
            pl.BlockSpec((d, d), lambda i: (0, 0)),
            pl.BlockSpec((1, d), lambda i: (0, 0)),
        ],
        out_specs=[
            pl.BlockSpec((BN, d), lambda i: (i, 0)),
            pl.BlockSpec((NC, BN, dh), lambda i: (0, i, 0)),
        ],
        out_shape=[
            jax.ShapeDtypeStruct((n, d), jnp.float32),
            jax.ShapeDtypeStruct((NC, n, dh), jnp.float32),
        ],
    )(partials, r, W_o1, b_o1.reshape(1, d), W_o2, b_o2.reshape(1, d),
      W_in, b_in.reshape(1, d))


def _tc_final(partials, r, W_o1, b_o1, W_o2, b_o2,
              W_r1, b_r1, w_r2_row, b_r2, n, BN=400):
    d = r.shape[1]
    dh = d // NC
    dr_h = W_r1.shape[1]

    def body(p_ref, r_ref, w1_ref, b1_ref, w2_ref, b2_ref,
             wr1_ref, br1_ref, wr2_ref, br2_ref, o_ref):
        i = pl.program_id(0)
        agg = jnp.concatenate([p_ref[0], p_ref[1]], axis=1)
        t = _ssp(_dot(agg, w1_ref[...]) + b1_ref[...])
        dr = _dot(t, w2_ref[...]) + b2_ref[...]
        rn = r_ref[...] + dr
        t2 = _ssp(_dot(rn, wr1_ref[...]) + br1_ref[...])
        t2b = t2.astype(jnp.bfloat16).astype(jnp.float32)
        w2b = wr2_ref[...].astype(jnp.bfloat16).astype(jnp.float32)
        s = jnp.sum(t2b * w2b) + BN * br2_ref[0, 0]

        @pl.when(i == 0)
        def _():
            o_ref[...] = jnp.zeros((1, 1), jnp.float32)

        o_ref[...] = o_ref[...] + s

    return pl.pallas_call(
        body,
        grid=(n // BN,),
        in_specs=[
            pl.BlockSpec((NC, BN, dh), lambda i: (0, i, 0)),
            pl.BlockSpec((BN, d), lambda i: (i, 0)),
            pl.BlockSpec((d, d), lambda i: (0, 0)),
            pl.BlockSpec((1, d), lambda i: (0, 0)),
            pl.BlockSpec((d, d), lambda i: (0, 0)),
            pl.BlockSpec((1, d), lambda i: (0, 0)),
            pl.BlockSpec((d, dr_h), lambda i: (0, 0)),
            pl.BlockSpec((1, dr_h), lambda i: (0, 0)),
            pl.BlockSpec((1, dr_h), lambda i: (0, 0)),
            pl.BlockSpec((1, 1), lambda i: (0, 0)),
        ],
        out_specs=pl.BlockSpec((1, 1), lambda i: (0, 0)),
        out_shape=jax.ShapeDtypeStruct((1, 1), jnp.float32),
    )(partials, r, W_o1, b_o1.reshape(1, d), W_o2, b_o2.reshape(1, d),
      W_r1, b_r1.reshape(1, dr_h), w_r2_row, b_r2.reshape(1, 1))


def kernel(z, xyz, nbr_list, num_atoms, embed,
           W_in2f, b_in2f, W_f1, b_f1, W_f2, b_f2,
           W_o1, b_o1, W_o2, b_o2, W_r1, b_r1, W_r2, b_r2):
    cutoff = 5.0
    n = z.shape[0]
    e = nbr_list.shape[0]
    c_layers = W_in2f.shape[0]
    n_gauss = W_f1.shape[1]
    d_feat = embed.shape[1]
    dh = d_feat // NC

    e_pad = -(-e // (NS * CHUNK * 2)) * (NS * CHUNK * 2)
    n_pad = -(-n // (NW * 64)) * (NW * 64)
    n_acc = -(-(n + 1) // (8 * NS)) * (8 * NS)

    nbr = nbr_list.astype(jnp.int32)
    src = jnp.concatenate([nbr[:, 0], jnp.full((e_pad - e,), n, jnp.int32)])
    dst = jnp.concatenate([nbr[:, 1], jnp.zeros((e_pad - e,), jnp.int32)])
    z_pad = jnp.pad(z.astype(jnp.int32), (0, n_pad - n))

    hWs = _tc_embed_table(embed, W_in2f[0], b_in2f[0])
    r0_pad, h2, d2 = _sc_prep(z_pad, xyz[:, 0], xyz[:, 1], xyz[:, 2],
                              src, dst, embed, hWs)
    r = r0_pad
    d2 = d2.reshape(e_pad, 1)

    gp = -(-n_gauss // 8) * 8
    W_f1p = jnp.pad(W_f1, ((0, 0), (0, gp - n_gauss), (0, 0)))
    w_all = _tc_filter(d2, W_f1p, b_f1, W_f2, b_f2, cutoff, n_gauss)

    n_chunks = e_pad // (NS * CHUNK)
    src3 = src.reshape(NS, n_chunks, CHUNK)
    dst3 = dst.reshape(NS, n_chunks, CHUNK)
    w_all6 = w_all.reshape(c_layers, NC, NS, n_chunks, CHUNK, dh)
    zeros_init = jnp.zeros((n_acc // NS, dh), jnp.float32)
    for i in range(c_layers):
        partials = _sc_message(h2, w_all6, i, src3, dst3, zeros_init, n_acc)
        if i + 1 < c_layers:
            r, h2 = _tc_update_dense(partials, r, W_o1[i], b_o1[i],
                                     W_o2[i], b_o2[i],
                                     W_in2f[i + 1], b_in2f[i + 1], n)
        else:
            energy = _tc_final(partials, r, W_o1[i], b_o1[i],
                               W_o2[i], b_o2[i],
                               W_r1, b_r1, W_r2.T, b_r2, n)

    return energy.reshape(1)
